# baseline (speedup 1.0000x reference)
import functools
import jax
import jax.numpy as jnp
from jax.experimental import pallas as pl
from jax.experimental.pallas import tpu as pltpu

_TM = 256


def _moe_dense_block(x_ref, wr_ref, w1_ref, w2_ref, w3_ref, o_ref):
    e = pl.program_id(1)
    x = x_ref[...]
    logits = jnp.dot(x, wr_ref[...].T, preferred_element_type=jnp.float32)
    s = jax.nn.softmax(logits, axis=-1)
    ne = s.shape[-1]
    iota = jax.lax.broadcasted_iota(jnp.int32, s.shape, 1)
    m1 = jnp.max(s, axis=-1, keepdims=True)
    i1 = jnp.min(jnp.where(s == m1, iota, ne), axis=-1, keepdims=True)
    s2 = jnp.where(iota == i1, -jnp.inf, s)
    m2 = jnp.max(s2, axis=-1, keepdims=True)
    i2 = jnp.min(jnp.where(s2 == m2, iota, ne), axis=-1, keepdims=True)
    gate = jnp.where(i1 == e, m1, 0.0) + jnp.where(i2 == e, m2, 0.0)

    a = jnp.dot(x, w1_ref[0], preferred_element_type=jnp.float32)
    b = jnp.dot(x, w3_ref[0], preferred_element_type=jnp.float32)
    h = (a * jax.nn.sigmoid(a)) * b
    y = jnp.dot(h, w2_ref[0], preferred_element_type=jnp.float32)

    @pl.when(e == 0)
    def _():
        o_ref[...] = jnp.zeros_like(o_ref)

    o_ref[...] += gate * y


def kernel(x, Wr, w1, w2, w3):
    bsz, seqlen, dim = x.shape
    T = bsz * seqlen
    E, hid = w1.shape[0], w1.shape[2]
    xf = x.reshape(T, dim)

    out = pl.pallas_call(
        _moe_dense_block,
        grid=(T // _TM, E),
        in_specs=[
            pl.BlockSpec((_TM, dim), lambda m, e: (m, 0)),
            pl.BlockSpec((E, dim), lambda m, e: (0, 0)),
            pl.BlockSpec((1, dim, hid), lambda m, e: (e, 0, 0)),
            pl.BlockSpec((1, hid, dim), lambda m, e: (e, 0, 0)),
            pl.BlockSpec((1, dim, hid), lambda m, e: (e, 0, 0)),
        ],
        out_specs=pl.BlockSpec((_TM, dim), lambda m, e: (m, 0)),
        out_shape=jax.ShapeDtypeStruct((T, dim), jnp.float32),
        compiler_params=pltpu.CompilerParams(
            dimension_semantics=("arbitrary", "arbitrary"),
        ),
    )(xf, Wr, w1, w2, w3)
    return out.reshape(bsz, seqlen, dim)


# speedup vs baseline: 1.3904x; 1.3904x over previous
import functools
import jax
import jax.numpy as jnp
from jax import lax
from jax.experimental import pallas as pl
from jax.experimental.pallas import tpu as pltpu
from jax.experimental.pallas import tpu_sc as plsc

_T = 2048
_DIM = 1024
_E = 8
_HID = 1408
_TM = 128
_NA = _T * 2
_NT = _NA // _TM
_NP = _E * _NT
_NS = 40
_NSC = 32
_TPW = _T // _NSC


def _router_body(x_ref, wr_ref, pos0_ref, pos1_ref, spos0_ref, spos1_ref,
                 wsort_ref, se_ref, sm_ref, slo_ref, shi_ref):
    f32 = jnp.float32
    x = x_ref[...]
    logits = lax.dot_general(x, wr_ref[...], (((1,), (1,)), ((), ())),
                             preferred_element_type=f32)
    z = logits - jnp.max(logits, axis=-1, keepdims=True)
    ez = jnp.exp(z)
    s = ez / jnp.sum(ez, axis=-1, keepdims=True)
    ei = lax.broadcasted_iota(jnp.int32, (_T, _E), 1)
    m1 = jnp.max(s, axis=-1, keepdims=True)
    i1 = jnp.min(jnp.where(s == m1, ei, _E), axis=-1, keepdims=True)
    s2 = jnp.where(ei == i1, -1.0, s)
    m2 = jnp.max(s2, axis=-1, keepdims=True)
    i2 = jnp.min(jnp.where(s2 == m2, ei, _E), axis=-1, keepdims=True)
    c1 = jnp.where(ei == i1, 1.0, 0.0)
    c2 = jnp.where(ei == i2, 1.0, 0.0)
    cc = c1 + c2

    ch_n = 256
    ti = lax.broadcasted_iota(jnp.int32, (ch_n, ch_n), 0)
    tj = lax.broadcasted_iota(jnp.int32, (ch_n, ch_n), 1)
    ltri = jnp.where(ti > tj, 1.0, 0.0)
    parts = []
    carry = jnp.zeros((1, _E), f32)
    for c in range(_T // ch_n):
        chk = lax.slice_in_dim(cc, c * ch_n, (c + 1) * ch_n, axis=0)
        parts.append(jnp.dot(ltri, chk, preferred_element_type=f32, precision=lax.Precision.HIGHEST) + carry)
        carry = carry + jnp.sum(chk, axis=0, keepdims=True)
    excl = jnp.concatenate(parts, axis=0)
    hist = carry
    e8i = lax.broadcasted_iota(jnp.int32, (_E, _E), 0)
    e8j = lax.broadcasted_iota(jnp.int32, (_E, _E), 1)
    su8 = jnp.where(e8i < e8j, 1.0, 0.0)
    off = jnp.dot(hist, su8, preferred_element_type=f32, precision=lax.Precision.HIGHEST)

    offc = off + excl
    pos0f = jnp.sum(offc * c1, axis=-1, keepdims=True)
    pos1f = jnp.sum(offc * c2, axis=-1, keepdims=True)
    pos0 = pos0f.astype(jnp.int32)
    pos1 = pos1f.astype(jnp.int32)
    pos0_ref[...] = pos0
    pos1_ref[...] = pos1

    q_c = lax.broadcasted_iota(jnp.int32, (_NP, 1), 0)
    e_qc = q_c // _NT
    m_qc = q_c % _NT
    ohe_c = jnp.where(e_qc == lax.broadcasted_iota(jnp.int32, (_NP, _E), 1),
                      1.0, 0.0)
    lo_c = lax.dot_general(ohe_c, off, (((1,), (1,)), ((), ())),
                           preferred_element_type=f32, precision=lax.Precision.HIGHEST)
    hist_c = lax.dot_general(ohe_c, hist, (((1,), (1,)), ((), ())),
                             preferred_element_type=f32, precision=lax.Precision.HIGHEST)
    hi_c = lo_c + hist_c
    tlo_c = (m_qc * _TM).astype(f32)
    thi_c = tlo_c + _TM
    valid_c = jnp.where(
        (lo_c < thi_c) & (hi_c > tlo_c) & (hist_c > 0.5), 1.0, 0.0)
    slo_c = jnp.maximum(lo_c, tlo_c)
    shi_c = jnp.minimum(hi_c, thi_c)
    qi = lax.broadcasted_iota(jnp.int32, (_NP, _NP), 0)
    qj = lax.broadcasted_iota(jnp.int32, (_NP, _NP), 1)
    ltq = jnp.where(qi > qj, 1.0, 0.0)
    idq = jnp.where(qi == qj, 1.0, 0.0)
    rank_c = jnp.dot(ltq, valid_c, preferred_element_type=f32, precision=lax.Precision.HIGHEST)
    rank_r = lax.dot_general(rank_c, idq, (((0,), (0,)), ((), ())),
                             preferred_element_type=f32, precision=lax.Precision.HIGHEST)
    valid_r = lax.dot_general(valid_c, idq, (((0,), (0,)), ((), ())),
                              preferred_element_type=f32, precision=lax.Precision.HIGHEST)
    ns = jnp.sum(valid_c)

    q0 = i1 * _NT + pos0 // _TM
    q1 = i2 * _NT + pos1 // _TM
    q_r = lax.broadcasted_iota(jnp.int32, (1, _NP), 1)
    oh0 = jnp.where(q0 == q_r, 1.0, 0.0)
    oh1 = jnp.where(q1 == q_r, 1.0, 0.0)
    rank0 = jnp.dot(oh0, rank_c, preferred_element_type=f32, precision=lax.Precision.HIGHEST)
    rank1 = jnp.dot(oh1, rank_c, preferred_element_type=f32, precision=lax.Precision.HIGHEST)
    spos0_ref[...] = rank0.astype(jnp.int32) * _TM + pos0 % _TM
    spos1_ref[...] = rank1.astype(jnp.int32) * _TM + pos1 % _TM

    wparts = []
    pc = 512
    for c in range(_NA // pc):
        p_r = lax.broadcasted_iota(jnp.int32, (1, pc), 1) + c * pc
        eq0 = jnp.where(pos0 == p_r, 1.0, 0.0)
        eq1 = jnp.where(pos1 == p_r, 1.0, 0.0)
        wc = (lax.dot_general(eq0, m1, (((0,), (0,)), ((), ())),
                              preferred_element_type=f32, precision=lax.Precision.HIGHEST) +
              lax.dot_general(eq1, m2, (((0,), (0,)), ((), ())),
                              preferred_element_type=f32, precision=lax.Precision.HIGHEST))
        wparts.append(wc)
    wsort_ref[...] = jnp.concatenate(wparts, axis=0)

    g_col = lax.broadcasted_iota(jnp.int32, (128, 1), 0).astype(f32)
    g_cl = jnp.minimum(g_col, jnp.maximum(ns - 1.0, 0.0))
    sel = jnp.where((rank_r == g_cl) & (valid_r > 0.5), 1.0, 0.0)
    se_ref[...] = jnp.dot(sel, e_qc.astype(f32),
                          preferred_element_type=f32, precision=lax.Precision.HIGHEST).astype(jnp.int32)
    sm_ref[...] = jnp.dot(sel, m_qc.astype(f32),
                          preferred_element_type=f32, precision=lax.Precision.HIGHEST).astype(jnp.int32)
    slo_ref[...] = jnp.dot(sel, slo_c,
                           preferred_element_type=f32, precision=lax.Precision.HIGHEST).astype(jnp.int32)
    shi_ref[...] = jnp.dot(sel, shi_c,
                           preferred_element_type=f32, precision=lax.Precision.HIGHEST).astype(jnp.int32)


def _router(xf, wr):
    i32 = jnp.int32
    outs = pl.pallas_call(
        _router_body,
        in_specs=[pl.BlockSpec(memory_space=pltpu.VMEM),
                  pl.BlockSpec(memory_space=pltpu.VMEM)],
        out_shape=[
            jax.ShapeDtypeStruct((_T, 1), i32),
            jax.ShapeDtypeStruct((_T, 1), i32),
            jax.ShapeDtypeStruct((_T, 1), i32),
            jax.ShapeDtypeStruct((_T, 1), i32),
            jax.ShapeDtypeStruct((_NA, 1), jnp.float32),
            jax.ShapeDtypeStruct((128, 1), i32),
            jax.ShapeDtypeStruct((128, 1), i32),
            jax.ShapeDtypeStruct((128, 1), i32),
            jax.ShapeDtypeStruct((128, 1), i32),
        ],
    )(xf, wr)
    return outs


def _gmm_body(se_ref, sm_ref, slo_ref, shi_ref,
              x_ref, wg_ref, w1_ref, w3_ref, w2_ref, o_ref):
    g = pl.program_id(0)
    lo = slo_ref[g]
    hi = shi_ref[g]
    m = sm_ref[g]
    r = m * _TM + lax.broadcasted_iota(jnp.int32, (_TM, 1), 0)
    valid = (r >= lo) & (r < hi)
    x = x_ref[...]
    a = jnp.dot(x, w1_ref[0], preferred_element_type=jnp.float32)
    b = jnp.dot(x, w3_ref[0], preferred_element_type=jnp.float32)
    h = (a * jax.nn.sigmoid(a)) * b * wg_ref[...]
    h = jnp.where(valid, h, 0.0)
    o_ref[...] = jnp.dot(h, w2_ref[0], preferred_element_type=jnp.float32)


def _gmm(se, sm, slo, shi, xs, wsort, w1, w2, w3):
    grid_spec = pltpu.PrefetchScalarGridSpec(
        num_scalar_prefetch=4,
        grid=(_NS,),
        in_specs=[
            pl.BlockSpec((_TM, _DIM), lambda g, se, sm, slo, shi: (sm[g], 0)),
            pl.BlockSpec((_TM, 1), lambda g, se, sm, slo, shi: (sm[g], 0)),
            pl.BlockSpec((1, _DIM, _HID),
                         lambda g, se, sm, slo, shi: (se[g], 0, 0)),
            pl.BlockSpec((1, _DIM, _HID),
                         lambda g, se, sm, slo, shi: (se[g], 0, 0)),
            pl.BlockSpec((1, _HID, _DIM),
                         lambda g, se, sm, slo, shi: (se[g], 0, 0)),
        ],
        out_specs=pl.BlockSpec((_TM, _DIM), lambda g, se, sm, slo, shi: (g, 0)),
    )
    return pl.pallas_call(
        _gmm_body,
        grid_spec=grid_spec,
        out_shape=jax.ShapeDtypeStruct((_NS * _TM, _DIM), jnp.float32),
        compiler_params=pltpu.CompilerParams(
            dimension_semantics=("arbitrary",),
        ),
    )(se, sm, slo, shi, xs, wsort, w1, w3, w2)


def _dispatch(xf, pos0, pos1):
    mesh = plsc.VectorSubcoreMesh(core_axis_name="c", subcore_axis_name="s")

    @functools.partial(
        pl.kernel,
        out_type=jax.ShapeDtypeStruct((_NA, _DIM), jnp.float32),
        mesh=mesh,
        scratch_types=[
            pltpu.VMEM((_TPW,), jnp.int32),
            pltpu.VMEM((_TPW,), jnp.int32),
            pltpu.VMEM((_TPW, _DIM), jnp.float32),
            pltpu.SemaphoreType.DMA,
            pltpu.SemaphoreType.DMA,
        ],
    )
    def body(x_hbm, p0_hbm, p1_hbm, xs_hbm, i0, i1, xv, sem0, sem1):
        wid = lax.axis_index("s") * 2 + lax.axis_index("c")
        t0 = wid * _TPW
        pltpu.sync_copy(p0_hbm.at[pl.ds(t0, _TPW)], i0)
        pltpu.sync_copy(p1_hbm.at[pl.ds(t0, _TPW)], i1)
        pltpu.sync_copy(x_hbm.at[pl.ds(t0, _TPW)], xv)
        c0 = pltpu.async_copy(xv, xs_hbm.at[i0], sem0)
        c1 = pltpu.async_copy(xv, xs_hbm.at[i1], sem1)
        c0.wait()
        c1.wait()

    return body(xf, pos0, pos1)


def _combine(ys, spos0, spos1):
    mesh = plsc.VectorSubcoreMesh(core_axis_name="c", subcore_axis_name="s")
    cw = 32

    @functools.partial(
        pl.kernel,
        out_type=jax.ShapeDtypeStruct((_T, _DIM), jnp.float32),
        mesh=mesh,
        scratch_types=[
            pltpu.VMEM((cw,), jnp.int32),
            pltpu.VMEM((cw,), jnp.int32),
            pltpu.VMEM((cw, _DIM), jnp.float32),
            pltpu.VMEM((cw, _DIM), jnp.float32),
            pltpu.VMEM((cw, _DIM), jnp.float32),
            pltpu.SemaphoreType.DMA,
            pltpu.SemaphoreType.DMA,
            pltpu.SemaphoreType.DMA,
        ],
    )
    def body(ys_hbm, sp0_hbm, sp1_hbm, o_hbm, i0, i1, b0, b1, oc, s0, s1, so):
        wid = lax.axis_index("s") * 2 + lax.axis_index("c")
        t0 = wid * _TPW
        for c in range(_TPW // cw):
            base = t0 + c * cw
            pltpu.sync_copy(sp0_hbm.at[pl.ds(base, cw)], i0)
            pltpu.sync_copy(sp1_hbm.at[pl.ds(base, cw)], i1)
            g0 = pltpu.async_copy(ys_hbm.at[i0], b0, s0)
            g1 = pltpu.async_copy(ys_hbm.at[i1], b1, s1)
            g0.wait()
            g1.wait()
            for j in range(cw):
                def add_body(k, _):
                    kb = k * 64
                    for u in range(4):
                        sl = pl.ds(kb + u * 16, 16)
                        oc[j, sl] = b0[j, sl] + b1[j, sl]
                    return 0
                lax.fori_loop(0, _DIM // 64, add_body, 0)
            pltpu.sync_copy(oc, o_hbm.at[pl.ds(base, cw)])

    return body(ys, spos0, spos1)


def kernel(x, Wr, w1, w2, w3):
    bsz, seqlen, dim = x.shape
    xf = x.reshape(_T, _DIM)
    pos0, pos1, spos0, spos1, wsort, se, sm, slo, shi = _router(xf, Wr)
    pos0 = pos0.reshape(_T)
    pos1 = pos1.reshape(_T)
    spos0 = spos0.reshape(_T)
    spos1 = spos1.reshape(_T)
    se = se.reshape(128)
    sm = sm.reshape(128)
    slo = slo.reshape(128)
    shi = shi.reshape(128)
    xs = _dispatch(xf, pos0, pos1)
    ys = _gmm(se, sm, slo, shi, xs, wsort, w1, w2, w3)
    out = _combine(ys, spos0, spos1)
    return out.reshape(bsz, seqlen, dim)


# speedup vs baseline: 1.4280x; 1.0270x over previous
import functools
import jax
import jax.numpy as jnp
from jax import lax
from jax.experimental import pallas as pl
from jax.experimental.pallas import tpu as pltpu
from jax.experimental.pallas import tpu_sc as plsc

_T = 2048
_DIM = 1024
_E = 8
_HID = 1408
_TM = 128
_NA = _T * 2
_NT = _NA // _TM
_NP = _E * _NT
_NS = 40
_NSC = 32
_TPW = _T // _NSC


def _router_body(x_ref, wr_ref, pos0_ref, pos1_ref, spos0_ref, spos1_ref,
                 wsort_ref, se_ref, sm_ref, slo_ref, shi_ref):
    f32 = jnp.float32
    x = x_ref[...]
    logits = lax.dot_general(x, wr_ref[...], (((1,), (1,)), ((), ())),
                             preferred_element_type=f32)
    z = logits - jnp.max(logits, axis=-1, keepdims=True)
    ez = jnp.exp(z)
    s = ez / jnp.sum(ez, axis=-1, keepdims=True)
    ei = lax.broadcasted_iota(jnp.int32, (_T, _E), 1)
    m1 = jnp.max(s, axis=-1, keepdims=True)
    i1 = jnp.min(jnp.where(s == m1, ei, _E), axis=-1, keepdims=True)
    s2 = jnp.where(ei == i1, -1.0, s)
    m2 = jnp.max(s2, axis=-1, keepdims=True)
    i2 = jnp.min(jnp.where(s2 == m2, ei, _E), axis=-1, keepdims=True)
    c1 = jnp.where(ei == i1, 1.0, 0.0)
    c2 = jnp.where(ei == i2, 1.0, 0.0)
    cc = c1 + c2

    ch_n = 256
    ti = lax.broadcasted_iota(jnp.int32, (ch_n, ch_n), 0)
    tj = lax.broadcasted_iota(jnp.int32, (ch_n, ch_n), 1)
    ltri = jnp.where(ti > tj, 1.0, 0.0)
    parts = []
    carry = jnp.zeros((1, _E), f32)
    for c in range(_T // ch_n):
        chk = lax.slice_in_dim(cc, c * ch_n, (c + 1) * ch_n, axis=0)
        parts.append(jnp.dot(ltri, chk, preferred_element_type=f32, precision=lax.Precision.HIGHEST) + carry)
        carry = carry + jnp.sum(chk, axis=0, keepdims=True)
    excl = jnp.concatenate(parts, axis=0)
    hist = carry
    e8i = lax.broadcasted_iota(jnp.int32, (_E, _E), 0)
    e8j = lax.broadcasted_iota(jnp.int32, (_E, _E), 1)
    su8 = jnp.where(e8i < e8j, 1.0, 0.0)
    off = jnp.dot(hist, su8, preferred_element_type=f32, precision=lax.Precision.HIGHEST)

    offc = off + excl
    pos0f = jnp.sum(offc * c1, axis=-1, keepdims=True)
    pos1f = jnp.sum(offc * c2, axis=-1, keepdims=True)
    pos0 = pos0f.astype(jnp.int32)
    pos1 = pos1f.astype(jnp.int32)
    pos0_ref[...] = pos0
    pos1_ref[...] = pos1

    q_c = lax.broadcasted_iota(jnp.int32, (_NP, 1), 0)
    e_qc = q_c // _NT
    m_qc = q_c % _NT
    ohe_c = jnp.where(e_qc == lax.broadcasted_iota(jnp.int32, (_NP, _E), 1),
                      1.0, 0.0)
    lo_c = lax.dot_general(ohe_c, off, (((1,), (1,)), ((), ())),
                           preferred_element_type=f32, precision=lax.Precision.HIGHEST)
    hist_c = lax.dot_general(ohe_c, hist, (((1,), (1,)), ((), ())),
                             preferred_element_type=f32, precision=lax.Precision.HIGHEST)
    hi_c = lo_c + hist_c
    tlo_c = (m_qc * _TM).astype(f32)
    thi_c = tlo_c + _TM
    valid_c = jnp.where(
        (lo_c < thi_c) & (hi_c > tlo_c) & (hist_c > 0.5), 1.0, 0.0)
    slo_c = jnp.maximum(lo_c, tlo_c)
    shi_c = jnp.minimum(hi_c, thi_c)
    qi = lax.broadcasted_iota(jnp.int32, (_NP, _NP), 0)
    qj = lax.broadcasted_iota(jnp.int32, (_NP, _NP), 1)
    ltq = jnp.where(qi > qj, 1.0, 0.0)
    idq = jnp.where(qi == qj, 1.0, 0.0)
    rank_c = jnp.dot(ltq, valid_c, preferred_element_type=f32, precision=lax.Precision.HIGHEST)
    rank_r = lax.dot_general(rank_c, idq, (((0,), (0,)), ((), ())),
                             preferred_element_type=f32, precision=lax.Precision.HIGHEST)
    valid_r = lax.dot_general(valid_c, idq, (((0,), (0,)), ((), ())),
                              preferred_element_type=f32, precision=lax.Precision.HIGHEST)
    ns = jnp.sum(valid_c)

    q0 = i1 * _NT + pos0 // _TM
    q1 = i2 * _NT + pos1 // _TM
    q_r = lax.broadcasted_iota(jnp.int32, (1, _NP), 1)
    oh0 = jnp.where(q0 == q_r, 1.0, 0.0)
    oh1 = jnp.where(q1 == q_r, 1.0, 0.0)
    rank0 = jnp.dot(oh0, rank_c, preferred_element_type=f32, precision=lax.Precision.HIGHEST)
    rank1 = jnp.dot(oh1, rank_c, preferred_element_type=f32, precision=lax.Precision.HIGHEST)
    spos0_ref[...] = rank0.astype(jnp.int32) * _TM + pos0 % _TM
    spos1_ref[...] = rank1.astype(jnp.int32) * _TM + pos1 % _TM

    wparts = []
    pc = 512
    for c in range(_NA // pc):
        p_r = lax.broadcasted_iota(jnp.int32, (1, pc), 1) + c * pc
        eq0 = jnp.where(pos0 == p_r, 1.0, 0.0)
        eq1 = jnp.where(pos1 == p_r, 1.0, 0.0)
        wc = (lax.dot_general(eq0, m1, (((0,), (0,)), ((), ())),
                              preferred_element_type=f32, precision=lax.Precision.HIGHEST) +
              lax.dot_general(eq1, m2, (((0,), (0,)), ((), ())),
                              preferred_element_type=f32, precision=lax.Precision.HIGHEST))
        wparts.append(wc)
    wsort_ref[...] = jnp.concatenate(wparts, axis=0)

    g_col = lax.broadcasted_iota(jnp.int32, (128, 1), 0).astype(f32)
    g_cl = jnp.minimum(g_col, jnp.maximum(ns - 1.0, 0.0))
    sel = jnp.where((rank_r == g_cl) & (valid_r > 0.5), 1.0, 0.0)
    se_ref[...] = jnp.dot(sel, e_qc.astype(f32),
                          preferred_element_type=f32, precision=lax.Precision.HIGHEST).astype(jnp.int32)
    sm_ref[...] = jnp.dot(sel, m_qc.astype(f32),
                          preferred_element_type=f32, precision=lax.Precision.HIGHEST).astype(jnp.int32)
    slo_ref[...] = jnp.dot(sel, slo_c,
                           preferred_element_type=f32, precision=lax.Precision.HIGHEST).astype(jnp.int32)
    shi_ref[...] = jnp.dot(sel, shi_c,
                           preferred_element_type=f32, precision=lax.Precision.HIGHEST).astype(jnp.int32)


def _router(xf, wr):
    i32 = jnp.int32
    outs = pl.pallas_call(
        _router_body,
        in_specs=[pl.BlockSpec(memory_space=pltpu.VMEM),
                  pl.BlockSpec(memory_space=pltpu.VMEM)],
        out_shape=[
            jax.ShapeDtypeStruct((_T, 1), i32),
            jax.ShapeDtypeStruct((_T, 1), i32),
            jax.ShapeDtypeStruct((_T, 1), i32),
            jax.ShapeDtypeStruct((_T, 1), i32),
            jax.ShapeDtypeStruct((_NA, 1), jnp.float32),
            jax.ShapeDtypeStruct((128, 1), i32),
            jax.ShapeDtypeStruct((128, 1), i32),
            jax.ShapeDtypeStruct((128, 1), i32),
            jax.ShapeDtypeStruct((128, 1), i32),
        ],
    )(xf, wr)
    return outs


def _gmm_body(se_ref, sm_ref, slo_ref, shi_ref,
              x_ref, wg_ref, w1_ref, w3_ref, w2_ref, o_ref):
    g = pl.program_id(0)
    lo = slo_ref[g]
    hi = shi_ref[g]
    m = sm_ref[g]
    r = m * _TM + lax.broadcasted_iota(jnp.int32, (_TM, 1), 0)
    valid = (r >= lo) & (r < hi)
    x = x_ref[...]
    a = jnp.dot(x, w1_ref[0], preferred_element_type=jnp.float32)
    b = jnp.dot(x, w3_ref[0], preferred_element_type=jnp.float32)
    h = (a * jax.nn.sigmoid(a)) * b * wg_ref[...]
    h = jnp.where(valid, h, 0.0)
    o_ref[...] = jnp.dot(h, w2_ref[0], preferred_element_type=jnp.float32)


def _gmm(se, sm, slo, shi, xs, wsort, w1, w2, w3):
    grid_spec = pltpu.PrefetchScalarGridSpec(
        num_scalar_prefetch=4,
        grid=(_NS,),
        in_specs=[
            pl.BlockSpec((_TM, _DIM), lambda g, se, sm, slo, shi: (sm[g], 0)),
            pl.BlockSpec((_TM, 1), lambda g, se, sm, slo, shi: (sm[g], 0)),
            pl.BlockSpec((1, _DIM, _HID),
                         lambda g, se, sm, slo, shi: (se[g], 0, 0)),
            pl.BlockSpec((1, _DIM, _HID),
                         lambda g, se, sm, slo, shi: (se[g], 0, 0)),
            pl.BlockSpec((1, _HID, _DIM),
                         lambda g, se, sm, slo, shi: (se[g], 0, 0)),
        ],
        out_specs=pl.BlockSpec((_TM, _DIM), lambda g, se, sm, slo, shi: (g, 0)),
    )
    return pl.pallas_call(
        _gmm_body,
        grid_spec=grid_spec,
        out_shape=jax.ShapeDtypeStruct((_NS * _TM, _DIM), jnp.float32),
        compiler_params=pltpu.CompilerParams(
            dimension_semantics=("arbitrary",),
        ),
    )(se, sm, slo, shi, xs, wsort, w1, w3, w2)


def _dispatch(xf, pos0, pos1):
    mesh = plsc.VectorSubcoreMesh(core_axis_name="c", subcore_axis_name="s")

    @functools.partial(
        pl.kernel,
        out_type=jax.ShapeDtypeStruct((_NA, _DIM), jnp.float32),
        mesh=mesh,
        scratch_types=[
            pltpu.VMEM((_TPW,), jnp.int32),
            pltpu.VMEM((_TPW,), jnp.int32),
            pltpu.VMEM((_TPW, _DIM), jnp.float32),
            pltpu.SemaphoreType.DMA,
            pltpu.SemaphoreType.DMA,
        ],
    )
    def body(x_hbm, p0_hbm, p1_hbm, xs_hbm, i0, i1, xv, sem0, sem1):
        wid = lax.axis_index("s") * 2 + lax.axis_index("c")
        t0 = wid * _TPW
        pltpu.sync_copy(p0_hbm.at[pl.ds(t0, _TPW)], i0)
        pltpu.sync_copy(p1_hbm.at[pl.ds(t0, _TPW)], i1)
        pltpu.sync_copy(x_hbm.at[pl.ds(t0, _TPW)], xv)
        c0 = pltpu.async_copy(xv, xs_hbm.at[i0], sem0)
        c1 = pltpu.async_copy(xv, xs_hbm.at[i1], sem1)
        c0.wait()
        c1.wait()

    return body(xf, pos0, pos1)


def _combine(ys, spos0, spos1):
    mesh = plsc.VectorSubcoreMesh(core_axis_name="c", subcore_axis_name="s")
    cw = 16
    nck = _TPW // cw

    @functools.partial(
        pl.kernel,
        out_type=jax.ShapeDtypeStruct((_T, _DIM), jnp.float32),
        mesh=mesh,
        scratch_types=[
            pltpu.VMEM((_TPW,), jnp.int32),
            pltpu.VMEM((_TPW,), jnp.int32),
            pltpu.VMEM((2, cw, _DIM), jnp.float32),
            pltpu.VMEM((2, cw, _DIM), jnp.float32),
            pltpu.VMEM((2, cw, _DIM), jnp.float32),
            pltpu.SemaphoreType.DMA((2,)),
            pltpu.SemaphoreType.DMA((2,)),
            pltpu.SemaphoreType.DMA((2,)),
        ],
    )
    def body(ys_hbm, sp0_hbm, sp1_hbm, o_hbm, i0, i1, b0, b1, oc, s0, s1, so):
        wid = lax.axis_index("s") * 2 + lax.axis_index("c")
        t0 = wid * _TPW
        pltpu.sync_copy(sp0_hbm.at[pl.ds(t0, _TPW)], i0)
        pltpu.sync_copy(sp1_hbm.at[pl.ds(t0, _TPW)], i1)

        def start_gather(c, sl):
            idx0 = i0[pl.ds(c * cw, cw)]
            idx1 = i1[pl.ds(c * cw, cw)]
            g0 = pltpu.make_async_copy(ys_hbm.at[idx0], b0.at[sl], s0.at[sl])
            g1 = pltpu.make_async_copy(ys_hbm.at[idx1], b1.at[sl], s1.at[sl])
            g0.start()
            g1.start()
            return g0, g1

        pend = start_gather(0, 0)
        wr = [None, None]
        for c in range(nck):
            sl = c % 2
            g0, g1 = pend
            if c + 1 < nck:
                nxt = start_gather(c + 1, (c + 1) % 2)
            g0.wait()
            g1.wait()
            if wr[sl] is not None:
                wr[sl].wait()
            for j in range(cw):
                def add_body(k, _):
                    kb = k * 64
                    for u in range(4):
                        ds = pl.ds(kb + u * 16, 16)
                        oc[sl, j, ds] = b0[sl, j, ds] + b1[sl, j, ds]
                    return 0
                lax.fori_loop(0, _DIM // 64, add_body, 0)
            w = pltpu.make_async_copy(
                oc.at[sl], o_hbm.at[pl.ds(t0 + c * cw, cw)], so.at[sl])
            w.start()
            wr[sl] = w
            if c + 1 < nck:
                pend = nxt
        for w in wr:
            if w is not None:
                w.wait()

    return body(ys, spos0, spos1)


def kernel(x, Wr, w1, w2, w3):
    bsz, seqlen, dim = x.shape
    xf = x.reshape(_T, _DIM)
    pos0, pos1, spos0, spos1, wsort, se, sm, slo, shi = _router(xf, Wr)
    pos0 = pos0.reshape(_T)
    pos1 = pos1.reshape(_T)
    spos0 = spos0.reshape(_T)
    spos1 = spos1.reshape(_T)
    se = se.reshape(128)
    sm = sm.reshape(128)
    slo = slo.reshape(128)
    shi = shi.reshape(128)
    xs = _dispatch(xf, pos0, pos1)
    ys = _gmm(se, sm, slo, shi, xs, wsort, w1, w2, w3)
    out = _combine(ys, spos0, spos1)
    return out.reshape(bsz, seqlen, dim)


# speedup vs baseline: 1.5890x; 1.1127x over previous
import functools
import jax
import jax.numpy as jnp
from jax import lax
from jax.experimental import pallas as pl
from jax.experimental.pallas import tpu as pltpu
from jax.experimental.pallas import tpu_sc as plsc

_T = 2048
_DIM = 1024
_E = 8
_HID = 1408
_TM = 128
_NA = _T * 2
_NT = _NA // _TM
_NP = _E * _NT
_NS = 40
_NSC = 32
_TPW = _T // _NSC


def _router_body(x_ref, wr_ref, pos0_ref, pos1_ref, spos0_ref, spos1_ref,
                 g0_ref, g1_ref, se_ref, sm_ref, slo_ref, shi_ref):
    f32 = jnp.float32
    x = x_ref[...]
    logits = lax.dot_general(x, wr_ref[...], (((1,), (1,)), ((), ())),
                             preferred_element_type=f32)
    z = logits - jnp.max(logits, axis=-1, keepdims=True)
    ez = jnp.exp(z)
    s = ez / jnp.sum(ez, axis=-1, keepdims=True)
    ei = lax.broadcasted_iota(jnp.int32, (_T, _E), 1)
    m1 = jnp.max(s, axis=-1, keepdims=True)
    i1 = jnp.min(jnp.where(s == m1, ei, _E), axis=-1, keepdims=True)
    s2 = jnp.where(ei == i1, -1.0, s)
    m2 = jnp.max(s2, axis=-1, keepdims=True)
    i2 = jnp.min(jnp.where(s2 == m2, ei, _E), axis=-1, keepdims=True)
    c1 = jnp.where(ei == i1, 1.0, 0.0)
    c2 = jnp.where(ei == i2, 1.0, 0.0)
    cc = c1 + c2

    ch_n = 256
    ti = lax.broadcasted_iota(jnp.int32, (ch_n, ch_n), 0)
    tj = lax.broadcasted_iota(jnp.int32, (ch_n, ch_n), 1)
    ltri = jnp.where(ti > tj, 1.0, 0.0)
    parts = []
    carry = jnp.zeros((1, _E), f32)
    for c in range(_T // ch_n):
        chk = lax.slice_in_dim(cc, c * ch_n, (c + 1) * ch_n, axis=0)
        parts.append(jnp.dot(ltri, chk, preferred_element_type=f32, precision=lax.Precision.HIGHEST) + carry)
        carry = carry + jnp.sum(chk, axis=0, keepdims=True)
    excl = jnp.concatenate(parts, axis=0)
    hist = carry
    e8i = lax.broadcasted_iota(jnp.int32, (_E, _E), 0)
    e8j = lax.broadcasted_iota(jnp.int32, (_E, _E), 1)
    su8 = jnp.where(e8i < e8j, 1.0, 0.0)
    off = jnp.dot(hist, su8, preferred_element_type=f32, precision=lax.Precision.HIGHEST)

    offc = off + excl
    pos0f = jnp.sum(offc * c1, axis=-1, keepdims=True)
    pos1f = jnp.sum(offc * c2, axis=-1, keepdims=True)
    pos0 = pos0f.astype(jnp.int32)
    pos1 = pos1f.astype(jnp.int32)
    pos0_ref[...] = pos0
    pos1_ref[...] = pos1

    q_c = lax.broadcasted_iota(jnp.int32, (_NP, 1), 0)
    e_qc = q_c // _NT
    m_qc = q_c % _NT
    ohe_c = jnp.where(e_qc == lax.broadcasted_iota(jnp.int32, (_NP, _E), 1),
                      1.0, 0.0)
    lo_c = lax.dot_general(ohe_c, off, (((1,), (1,)), ((), ())),
                           preferred_element_type=f32, precision=lax.Precision.HIGHEST)
    hist_c = lax.dot_general(ohe_c, hist, (((1,), (1,)), ((), ())),
                             preferred_element_type=f32, precision=lax.Precision.HIGHEST)
    hi_c = lo_c + hist_c
    tlo_c = (m_qc * _TM).astype(f32)
    thi_c = tlo_c + _TM
    valid_c = jnp.where(
        (lo_c < thi_c) & (hi_c > tlo_c) & (hist_c > 0.5), 1.0, 0.0)
    slo_c = jnp.maximum(lo_c, tlo_c)
    shi_c = jnp.minimum(hi_c, thi_c)
    qi = lax.broadcasted_iota(jnp.int32, (_NP, _NP), 0)
    qj = lax.broadcasted_iota(jnp.int32, (_NP, _NP), 1)
    ltq = jnp.where(qi > qj, 1.0, 0.0)
    idq = jnp.where(qi == qj, 1.0, 0.0)
    rank_c = jnp.dot(ltq, valid_c, preferred_element_type=f32, precision=lax.Precision.HIGHEST)
    rank_r = lax.dot_general(rank_c, idq, (((0,), (0,)), ((), ())),
                             preferred_element_type=f32, precision=lax.Precision.HIGHEST)
    valid_r = lax.dot_general(valid_c, idq, (((0,), (0,)), ((), ())),
                              preferred_element_type=f32, precision=lax.Precision.HIGHEST)
    ns = jnp.sum(valid_c)

    q0 = i1 * _NT + pos0 // _TM
    q1 = i2 * _NT + pos1 // _TM
    q_r = lax.broadcasted_iota(jnp.int32, (1, _NP), 1)
    oh0 = jnp.where(q0 == q_r, 1.0, 0.0)
    oh1 = jnp.where(q1 == q_r, 1.0, 0.0)
    rank0 = jnp.dot(oh0, rank_c, preferred_element_type=f32, precision=lax.Precision.HIGHEST)
    rank1 = jnp.dot(oh1, rank_c, preferred_element_type=f32, precision=lax.Precision.HIGHEST)
    spos0_ref[...] = rank0.astype(jnp.int32) * _TM + pos0 % _TM
    spos1_ref[...] = rank1.astype(jnp.int32) * _TM + pos1 % _TM

    g0_ref[...] = m1
    g1_ref[...] = m2

    g_col = lax.broadcasted_iota(jnp.int32, (128, 1), 0).astype(f32)
    g_cl = jnp.minimum(g_col, jnp.maximum(ns - 1.0, 0.0))
    sel = jnp.where((rank_r == g_cl) & (valid_r > 0.5), 1.0, 0.0)
    se_ref[...] = jnp.dot(sel, e_qc.astype(f32),
                          preferred_element_type=f32, precision=lax.Precision.HIGHEST).astype(jnp.int32)
    sm_ref[...] = jnp.dot(sel, m_qc.astype(f32),
                          preferred_element_type=f32, precision=lax.Precision.HIGHEST).astype(jnp.int32)
    slo_ref[...] = jnp.dot(sel, slo_c,
                           preferred_element_type=f32, precision=lax.Precision.HIGHEST).astype(jnp.int32)
    shi_ref[...] = jnp.dot(sel, shi_c,
                           preferred_element_type=f32, precision=lax.Precision.HIGHEST).astype(jnp.int32)


def _router(xf, wr):
    i32 = jnp.int32
    outs = pl.pallas_call(
        _router_body,
        in_specs=[pl.BlockSpec(memory_space=pltpu.VMEM),
                  pl.BlockSpec(memory_space=pltpu.VMEM)],
        out_shape=[
            jax.ShapeDtypeStruct((_T, 1), i32),
            jax.ShapeDtypeStruct((_T, 1), i32),
            jax.ShapeDtypeStruct((_T, 1), i32),
            jax.ShapeDtypeStruct((_T, 1), i32),
            jax.ShapeDtypeStruct((_T, 1), jnp.float32),
            jax.ShapeDtypeStruct((_T, 1), jnp.float32),
            jax.ShapeDtypeStruct((128, 1), i32),
            jax.ShapeDtypeStruct((128, 1), i32),
            jax.ShapeDtypeStruct((128, 1), i32),
            jax.ShapeDtypeStruct((128, 1), i32),
        ],
    )(xf, wr)
    return outs


def _gmm_body(se_ref, sm_ref, slo_ref, shi_ref,
              x_ref, wg_ref, w1_ref, w3_ref, w2_ref, o_ref):
    g = pl.program_id(0)
    lo = slo_ref[g]
    hi = shi_ref[g]
    m = sm_ref[g]
    r = m * _TM + lax.broadcasted_iota(jnp.int32, (_TM, 1), 0)
    valid = (r >= lo) & (r < hi)
    x = x_ref[...]
    a = jnp.dot(x, w1_ref[0], preferred_element_type=jnp.float32)
    b = jnp.dot(x, w3_ref[0], preferred_element_type=jnp.float32)
    h = (a * jax.nn.sigmoid(a)) * b * wg_ref[...]
    h = jnp.where(valid, h, 0.0)
    o_ref[...] = jnp.dot(h, w2_ref[0], preferred_element_type=jnp.float32)


def _gmm(se, sm, slo, shi, xs, wsort, w1, w2, w3):
    grid_spec = pltpu.PrefetchScalarGridSpec(
        num_scalar_prefetch=4,
        grid=(_NS,),
        in_specs=[
            pl.BlockSpec((_TM, _DIM), lambda g, se, sm, slo, shi: (sm[g], 0)),
            pl.BlockSpec((_TM, 1), lambda g, se, sm, slo, shi: (sm[g], 0)),
            pl.BlockSpec((1, _DIM, _HID),
                         lambda g, se, sm, slo, shi: (se[g], 0, 0)),
            pl.BlockSpec((1, _DIM, _HID),
                         lambda g, se, sm, slo, shi: (se[g], 0, 0)),
            pl.BlockSpec((1, _HID, _DIM),
                         lambda g, se, sm, slo, shi: (se[g], 0, 0)),
        ],
        out_specs=pl.BlockSpec((_TM, _DIM), lambda g, se, sm, slo, shi: (g, 0)),
    )
    return pl.pallas_call(
        _gmm_body,
        grid_spec=grid_spec,
        out_shape=jax.ShapeDtypeStruct((_NS * _TM, _DIM), jnp.float32),
        compiler_params=pltpu.CompilerParams(
            dimension_semantics=("arbitrary",),
        ),
    )(se, sm, slo, shi, xs, wsort, w1, w3, w2)


def _dispatch(xf, pos0, pos1, g0, g1):
    mesh = plsc.VectorSubcoreMesh(core_axis_name="c", subcore_axis_name="s")

    @functools.partial(
        pl.kernel,
        out_type=[jax.ShapeDtypeStruct((_NA, _DIM), jnp.float32),
                  jax.ShapeDtypeStruct((_NA,), jnp.float32)],
        mesh=mesh,
        scratch_types=[
            pltpu.VMEM((_TPW,), jnp.int32),
            pltpu.VMEM((_TPW,), jnp.int32),
            pltpu.VMEM((_TPW,), jnp.float32),
            pltpu.VMEM((_TPW,), jnp.float32),
            pltpu.VMEM((_TPW, _DIM), jnp.float32),
            pltpu.SemaphoreType.DMA,
            pltpu.SemaphoreType.DMA,
            pltpu.SemaphoreType.DMA,
            pltpu.SemaphoreType.DMA,
        ],
    )
    def body(x_hbm, p0_hbm, p1_hbm, g0_hbm, g1_hbm, xs_hbm, ws_hbm,
             i0, i1, gv0, gv1, xv, sem0, sem1, sem2, sem3):
        wid = lax.axis_index("s") * 2 + lax.axis_index("c")
        t0 = wid * _TPW
        pltpu.sync_copy(p0_hbm.at[pl.ds(t0, _TPW)], i0)
        pltpu.sync_copy(p1_hbm.at[pl.ds(t0, _TPW)], i1)
        pltpu.sync_copy(g0_hbm.at[pl.ds(t0, _TPW)], gv0)
        pltpu.sync_copy(g1_hbm.at[pl.ds(t0, _TPW)], gv1)
        pltpu.sync_copy(x_hbm.at[pl.ds(t0, _TPW)], xv)
        c0 = pltpu.async_copy(xv, xs_hbm.at[i0], sem0)
        c1 = pltpu.async_copy(xv, xs_hbm.at[i1], sem1)
        c2 = pltpu.async_copy(gv0, ws_hbm.at[i0], sem2)
        c3 = pltpu.async_copy(gv1, ws_hbm.at[i1], sem3)
        c0.wait()
        c1.wait()
        c2.wait()
        c3.wait()

    return body(xf, pos0, pos1, g0, g1)


def _combine(ys, spos0, spos1):
    mesh = plsc.VectorSubcoreMesh(core_axis_name="c", subcore_axis_name="s")
    cw = 16
    nck = _TPW // cw

    @functools.partial(
        pl.kernel,
        out_type=jax.ShapeDtypeStruct((_T, _DIM), jnp.float32),
        mesh=mesh,
        scratch_types=[
            pltpu.VMEM((_TPW,), jnp.int32),
            pltpu.VMEM((_TPW,), jnp.int32),
            pltpu.VMEM((2, cw, _DIM), jnp.float32),
            pltpu.VMEM((2, cw, _DIM), jnp.float32),
            pltpu.VMEM((2, cw, _DIM), jnp.float32),
            pltpu.SemaphoreType.DMA((2,)),
            pltpu.SemaphoreType.DMA((2,)),
            pltpu.SemaphoreType.DMA((2,)),
        ],
    )
    def body(ys_hbm, sp0_hbm, sp1_hbm, o_hbm, i0, i1, b0, b1, oc, s0, s1, so):
        wid = lax.axis_index("s") * 2 + lax.axis_index("c")
        t0 = wid * _TPW
        pltpu.sync_copy(sp0_hbm.at[pl.ds(t0, _TPW)], i0)
        pltpu.sync_copy(sp1_hbm.at[pl.ds(t0, _TPW)], i1)

        def start_gather(c, sl):
            idx0 = i0[pl.ds(c * cw, cw)]
            idx1 = i1[pl.ds(c * cw, cw)]
            g0 = pltpu.make_async_copy(ys_hbm.at[idx0], b0.at[sl], s0.at[sl])
            g1 = pltpu.make_async_copy(ys_hbm.at[idx1], b1.at[sl], s1.at[sl])
            g0.start()
            g1.start()
            return g0, g1

        pend = start_gather(0, 0)
        wr = [None, None]
        for c in range(nck):
            sl = c % 2
            g0, g1 = pend
            if c + 1 < nck:
                nxt = start_gather(c + 1, (c + 1) % 2)
            g0.wait()
            g1.wait()
            if wr[sl] is not None:
                wr[sl].wait()
            for j in range(cw):
                def add_body(k, _):
                    kb = k * 64
                    for u in range(4):
                        ds = pl.ds(kb + u * 16, 16)
                        oc[sl, j, ds] = b0[sl, j, ds] + b1[sl, j, ds]
                    return 0
                lax.fori_loop(0, _DIM // 64, add_body, 0)
            w = pltpu.make_async_copy(
                oc.at[sl], o_hbm.at[pl.ds(t0 + c * cw, cw)], so.at[sl])
            w.start()
            wr[sl] = w
            if c + 1 < nck:
                pend = nxt
        for w in wr:
            if w is not None:
                w.wait()

    return body(ys, spos0, spos1)


def kernel(x, Wr, w1, w2, w3):
    bsz, seqlen, dim = x.shape
    xf = x.reshape(_T, _DIM)
    pos0, pos1, spos0, spos1, g0, g1, se, sm, slo, shi = _router(xf, Wr)
    pos0 = pos0.reshape(_T)
    pos1 = pos1.reshape(_T)
    spos0 = spos0.reshape(_T)
    spos1 = spos1.reshape(_T)
    g0 = g0.reshape(_T)
    g1 = g1.reshape(_T)
    se = se.reshape(128)
    sm = sm.reshape(128)
    slo = slo.reshape(128)
    shi = shi.reshape(128)
    xs, ws = _dispatch(xf, pos0, pos1, g0, g1)
    wsort = ws.reshape(_NA, 1)
    ys = _gmm(se, sm, slo, shi, xs, wsort, w1, w2, w3)
    out = _combine(ys, spos0, spos1)
    return out.reshape(bsz, seqlen, dim)


# speedup vs baseline: 1.6329x; 1.0277x over previous
import functools
import jax
import jax.numpy as jnp
from jax import lax
from jax.experimental import pallas as pl
from jax.experimental.pallas import tpu as pltpu
from jax.experimental.pallas import tpu_sc as plsc

_T = 2048
_DIM = 1024
_E = 8
_HID = 1408
_TM = 128
_NA = _T * 2
_NT = _NA // _TM
_NP = _E * _NT
_NS = 40
_NSC = 32
_TPW = _T // _NSC


def _router_body(x_ref, wr_ref, pos0_ref, pos1_ref, spos0_ref, spos1_ref,
                 g0_ref, g1_ref, se_ref, sm_ref, slo_ref, shi_ref):
    f32 = jnp.float32
    x = x_ref[...]
    logits = lax.dot_general(x, wr_ref[...], (((1,), (1,)), ((), ())),
                             preferred_element_type=f32)
    z = logits - jnp.max(logits, axis=-1, keepdims=True)
    ez = jnp.exp(z)
    s = ez / jnp.sum(ez, axis=-1, keepdims=True)
    ei = lax.broadcasted_iota(jnp.int32, (_T, _E), 1)
    m1 = jnp.max(s, axis=-1, keepdims=True)
    i1 = jnp.min(jnp.where(s == m1, ei, _E), axis=-1, keepdims=True)
    s2 = jnp.where(ei == i1, -1.0, s)
    m2 = jnp.max(s2, axis=-1, keepdims=True)
    i2 = jnp.min(jnp.where(s2 == m2, ei, _E), axis=-1, keepdims=True)
    c1 = jnp.where(ei == i1, 1.0, 0.0)
    c2 = jnp.where(ei == i2, 1.0, 0.0)
    cc = c1 + c2

    ch_n = 256
    ti = lax.broadcasted_iota(jnp.int32, (ch_n, ch_n), 0)
    tj = lax.broadcasted_iota(jnp.int32, (ch_n, ch_n), 1)
    ltri = jnp.where(ti > tj, 1.0, 0.0)
    parts = []
    carry = jnp.zeros((1, _E), f32)
    for c in range(_T // ch_n):
        chk = lax.slice_in_dim(cc, c * ch_n, (c + 1) * ch_n, axis=0)
        parts.append(jnp.dot(ltri, chk, preferred_element_type=f32) + carry)
        carry = carry + jnp.sum(chk, axis=0, keepdims=True)
    excl = jnp.concatenate(parts, axis=0)
    hist = carry
    e8i = lax.broadcasted_iota(jnp.int32, (_E, _E), 0)
    e8j = lax.broadcasted_iota(jnp.int32, (_E, _E), 1)
    su8 = jnp.where(e8i < e8j, 1.0, 0.0)
    off = jnp.dot(hist, su8, preferred_element_type=f32, precision=lax.Precision.HIGHEST)

    offc = off + excl
    pos0f = jnp.sum(offc * c1, axis=-1, keepdims=True)
    pos1f = jnp.sum(offc * c2, axis=-1, keepdims=True)
    pos0 = pos0f.astype(jnp.int32)
    pos1 = pos1f.astype(jnp.int32)
    pos0_ref[...] = pos0
    pos1_ref[...] = pos1

    q_c = lax.broadcasted_iota(jnp.int32, (_NP, 1), 0)
    e_qc = q_c // _NT
    m_qc = q_c % _NT
    ohe_c = jnp.where(e_qc == lax.broadcasted_iota(jnp.int32, (_NP, _E), 1),
                      1.0, 0.0)
    lo_c = lax.dot_general(ohe_c, off, (((1,), (1,)), ((), ())),
                           preferred_element_type=f32, precision=lax.Precision.HIGHEST)
    hist_c = lax.dot_general(ohe_c, hist, (((1,), (1,)), ((), ())),
                             preferred_element_type=f32, precision=lax.Precision.HIGHEST)
    hi_c = lo_c + hist_c
    tlo_c = (m_qc * _TM).astype(f32)
    thi_c = tlo_c + _TM
    valid_c = jnp.where(
        (lo_c < thi_c) & (hi_c > tlo_c) & (hist_c > 0.5), 1.0, 0.0)
    slo_c = jnp.maximum(lo_c, tlo_c)
    shi_c = jnp.minimum(hi_c, thi_c)
    qi = lax.broadcasted_iota(jnp.int32, (_NP, _NP), 0)
    qj = lax.broadcasted_iota(jnp.int32, (_NP, _NP), 1)
    ltq = jnp.where(qi > qj, 1.0, 0.0)
    idq = jnp.where(qi == qj, 1.0, 0.0)
    rank_c = jnp.dot(ltq, valid_c, preferred_element_type=f32)
    rank_r = lax.dot_general(rank_c, idq, (((0,), (0,)), ((), ())),
                             preferred_element_type=f32)
    valid_r = lax.dot_general(valid_c, idq, (((0,), (0,)), ((), ())),
                              preferred_element_type=f32)
    ns = jnp.sum(valid_c)

    q0 = i1 * _NT + pos0 // _TM
    q1 = i2 * _NT + pos1 // _TM
    q_r = lax.broadcasted_iota(jnp.int32, (1, _NP), 1)
    oh0 = jnp.where(q0 == q_r, 1.0, 0.0)
    oh1 = jnp.where(q1 == q_r, 1.0, 0.0)
    rank0 = jnp.dot(oh0, rank_c, preferred_element_type=f32)
    rank1 = jnp.dot(oh1, rank_c, preferred_element_type=f32)
    spos0_ref[...] = rank0.astype(jnp.int32) * _TM + pos0 % _TM
    spos1_ref[...] = rank1.astype(jnp.int32) * _TM + pos1 % _TM

    g0_ref[...] = m1
    g1_ref[...] = m2

    g_col = lax.broadcasted_iota(jnp.int32, (128, 1), 0).astype(f32)
    g_cl = jnp.minimum(g_col, jnp.maximum(ns - 1.0, 0.0))
    sel = jnp.where((rank_r == g_cl) & (valid_r > 0.5), 1.0, 0.0)
    se_ref[...] = jnp.dot(sel, e_qc.astype(f32),
                          preferred_element_type=f32).astype(jnp.int32)
    sm_ref[...] = jnp.dot(sel, m_qc.astype(f32),
                          preferred_element_type=f32).astype(jnp.int32)
    slo_ref[...] = jnp.dot(sel, slo_c,
                           preferred_element_type=f32, precision=lax.Precision.HIGHEST).astype(jnp.int32)
    shi_ref[...] = jnp.dot(sel, shi_c,
                           preferred_element_type=f32, precision=lax.Precision.HIGHEST).astype(jnp.int32)


def _router(xf, wr):
    i32 = jnp.int32
    outs = pl.pallas_call(
        _router_body,
        in_specs=[pl.BlockSpec(memory_space=pltpu.VMEM),
                  pl.BlockSpec(memory_space=pltpu.VMEM)],
        out_shape=[
            jax.ShapeDtypeStruct((_T, 1), i32),
            jax.ShapeDtypeStruct((_T, 1), i32),
            jax.ShapeDtypeStruct((_T, 1), i32),
            jax.ShapeDtypeStruct((_T, 1), i32),
            jax.ShapeDtypeStruct((_T, 1), jnp.float32),
            jax.ShapeDtypeStruct((_T, 1), jnp.float32),
            jax.ShapeDtypeStruct((128, 1), i32),
            jax.ShapeDtypeStruct((128, 1), i32),
            jax.ShapeDtypeStruct((128, 1), i32),
            jax.ShapeDtypeStruct((128, 1), i32),
        ],
    )(xf, wr)
    return outs


def _gmm_body(se_ref, sm_ref, slo_ref, shi_ref,
              x_ref, wg_ref, w1_ref, w3_ref, w2_ref, o_ref):
    g = pl.program_id(0)
    lo = slo_ref[g]
    hi = shi_ref[g]
    m = sm_ref[g]
    r = m * _TM + lax.broadcasted_iota(jnp.int32, (_TM, 1), 0)
    valid = (r >= lo) & (r < hi)
    x = x_ref[...]
    a = jnp.dot(x, w1_ref[0], preferred_element_type=jnp.float32)
    b = jnp.dot(x, w3_ref[0], preferred_element_type=jnp.float32)
    h = (a * jax.nn.sigmoid(a)) * b * wg_ref[...]
    h = jnp.where(valid, h, 0.0)
    o_ref[...] = jnp.dot(h, w2_ref[0], preferred_element_type=jnp.float32)


def _gmm(se, sm, slo, shi, xs, wsort, w1, w2, w3):
    grid_spec = pltpu.PrefetchScalarGridSpec(
        num_scalar_prefetch=4,
        grid=(_NS,),
        in_specs=[
            pl.BlockSpec((_TM, _DIM), lambda g, se, sm, slo, shi: (sm[g], 0)),
            pl.BlockSpec((_TM, 1), lambda g, se, sm, slo, shi: (sm[g], 0)),
            pl.BlockSpec((1, _DIM, _HID),
                         lambda g, se, sm, slo, shi: (se[g], 0, 0)),
            pl.BlockSpec((1, _DIM, _HID),
                         lambda g, se, sm, slo, shi: (se[g], 0, 0)),
            pl.BlockSpec((1, _HID, _DIM),
                         lambda g, se, sm, slo, shi: (se[g], 0, 0)),
        ],
        out_specs=pl.BlockSpec((_TM, _DIM), lambda g, se, sm, slo, shi: (g, 0)),
    )
    return pl.pallas_call(
        _gmm_body,
        grid_spec=grid_spec,
        out_shape=jax.ShapeDtypeStruct((_NS * _TM, _DIM), jnp.float32),
        compiler_params=pltpu.CompilerParams(
            dimension_semantics=("arbitrary",),
        ),
    )(se, sm, slo, shi, xs, wsort, w1, w3, w2)


def _dispatch(xf, pos0, pos1, g0, g1):
    mesh = plsc.VectorSubcoreMesh(core_axis_name="c", subcore_axis_name="s")

    @functools.partial(
        pl.kernel,
        out_type=[jax.ShapeDtypeStruct((_NA, _DIM), jnp.float32),
                  jax.ShapeDtypeStruct((_NA,), jnp.float32)],
        mesh=mesh,
        scratch_types=[
            pltpu.VMEM((_TPW,), jnp.int32),
            pltpu.VMEM((_TPW,), jnp.int32),
            pltpu.VMEM((_TPW,), jnp.float32),
            pltpu.VMEM((_TPW,), jnp.float32),
            pltpu.VMEM((_TPW, _DIM), jnp.float32),
            pltpu.SemaphoreType.DMA,
            pltpu.SemaphoreType.DMA,
            pltpu.SemaphoreType.DMA,
            pltpu.SemaphoreType.DMA,
        ],
    )
    def body(x_hbm, p0_hbm, p1_hbm, g0_hbm, g1_hbm, xs_hbm, ws_hbm,
             i0, i1, gv0, gv1, xv, sem0, sem1, sem2, sem3):
        wid = lax.axis_index("s") * 2 + lax.axis_index("c")
        t0 = wid * _TPW
        pltpu.sync_copy(p0_hbm.at[pl.ds(t0, _TPW)], i0)
        pltpu.sync_copy(p1_hbm.at[pl.ds(t0, _TPW)], i1)
        pltpu.sync_copy(g0_hbm.at[pl.ds(t0, _TPW)], gv0)
        pltpu.sync_copy(g1_hbm.at[pl.ds(t0, _TPW)], gv1)
        pltpu.sync_copy(x_hbm.at[pl.ds(t0, _TPW)], xv)
        c0 = pltpu.async_copy(xv, xs_hbm.at[i0], sem0)
        c1 = pltpu.async_copy(xv, xs_hbm.at[i1], sem1)
        c2 = pltpu.async_copy(gv0, ws_hbm.at[i0], sem2)
        c3 = pltpu.async_copy(gv1, ws_hbm.at[i1], sem3)
        c0.wait()
        c1.wait()
        c2.wait()
        c3.wait()

    return body(xf, pos0, pos1, g0, g1)


def _combine(ys, spos0, spos1):
    mesh = plsc.VectorSubcoreMesh(core_axis_name="c", subcore_axis_name="s")
    cw = 16
    nck = _TPW // cw

    @functools.partial(
        pl.kernel,
        out_type=jax.ShapeDtypeStruct((_T, _DIM), jnp.float32),
        mesh=mesh,
        scratch_types=[
            pltpu.VMEM((_TPW,), jnp.int32),
            pltpu.VMEM((_TPW,), jnp.int32),
            pltpu.VMEM((2, cw, _DIM), jnp.float32),
            pltpu.VMEM((2, cw, _DIM), jnp.float32),
            pltpu.VMEM((2, cw, _DIM), jnp.float32),
            pltpu.SemaphoreType.DMA((2,)),
            pltpu.SemaphoreType.DMA((2,)),
            pltpu.SemaphoreType.DMA((2,)),
        ],
    )
    def body(ys_hbm, sp0_hbm, sp1_hbm, o_hbm, i0, i1, b0, b1, oc, s0, s1, so):
        wid = lax.axis_index("s") * 2 + lax.axis_index("c")
        t0 = wid * _TPW
        pltpu.sync_copy(sp0_hbm.at[pl.ds(t0, _TPW)], i0)
        pltpu.sync_copy(sp1_hbm.at[pl.ds(t0, _TPW)], i1)

        def start_gather(c, sl):
            idx0 = i0[pl.ds(c * cw, cw)]
            idx1 = i1[pl.ds(c * cw, cw)]
            g0 = pltpu.make_async_copy(ys_hbm.at[idx0], b0.at[sl], s0.at[sl])
            g1 = pltpu.make_async_copy(ys_hbm.at[idx1], b1.at[sl], s1.at[sl])
            g0.start()
            g1.start()
            return g0, g1

        pend = start_gather(0, 0)
        wr = [None, None]
        for c in range(nck):
            sl = c % 2
            g0, g1 = pend
            if c + 1 < nck:
                nxt = start_gather(c + 1, (c + 1) % 2)
            g0.wait()
            g1.wait()
            if wr[sl] is not None:
                wr[sl].wait()
            for j in range(cw):
                def add_body(k, _):
                    kb = k * 64
                    for u in range(4):
                        ds = pl.ds(kb + u * 16, 16)
                        oc[sl, j, ds] = b0[sl, j, ds] + b1[sl, j, ds]
                    return 0
                lax.fori_loop(0, _DIM // 64, add_body, 0)
            w = pltpu.make_async_copy(
                oc.at[sl], o_hbm.at[pl.ds(t0 + c * cw, cw)], so.at[sl])
            w.start()
            wr[sl] = w
            if c + 1 < nck:
                pend = nxt
        for w in wr:
            if w is not None:
                w.wait()

    return body(ys, spos0, spos1)


def kernel(x, Wr, w1, w2, w3):
    bsz, seqlen, dim = x.shape
    xf = x.reshape(_T, _DIM)
    pos0, pos1, spos0, spos1, g0, g1, se, sm, slo, shi = _router(xf, Wr)
    pos0 = pos0.reshape(_T)
    pos1 = pos1.reshape(_T)
    spos0 = spos0.reshape(_T)
    spos1 = spos1.reshape(_T)
    g0 = g0.reshape(_T)
    g1 = g1.reshape(_T)
    se = se.reshape(128)
    sm = sm.reshape(128)
    slo = slo.reshape(128)
    shi = shi.reshape(128)
    xs, ws = _dispatch(xf, pos0, pos1, g0, g1)
    wsort = ws.reshape(_NA, 1)
    ys = _gmm(se, sm, slo, shi, xs, wsort, w1, w2, w3)
    out = _combine(ys, spos0, spos1)
    return out.reshape(bsz, seqlen, dim)


# speedup vs baseline: 1.9248x; 1.1787x over previous
import functools
import jax
import jax.numpy as jnp
from jax import lax
from jax.experimental import pallas as pl
from jax.experimental.pallas import tpu as pltpu
from jax.experimental.pallas import tpu_sc as plsc

_T = 2048
_DIM = 1024
_E = 8
_HID = 1408
_TM = 128
_NA = _T * 2
_NT = _NA // _TM
_NP = _E * _NT
_NS = 40
_NSC = 32
_TPW = _T // _NSC


def _router_body(x_ref, wr_ref, pos0_ref, pos1_ref, spos0_ref, spos1_ref,
                 g0_ref, g1_ref, se_ref, sm_ref, slo_ref, shi_ref):
    f32 = jnp.float32
    x = x_ref[...]
    logits = lax.dot_general(x, wr_ref[...], (((1,), (1,)), ((), ())),
                             preferred_element_type=f32)
    z = logits - jnp.max(logits, axis=-1, keepdims=True)
    ez = jnp.exp(z)
    s = ez / jnp.sum(ez, axis=-1, keepdims=True)
    ei = lax.broadcasted_iota(jnp.int32, (_T, _E), 1)
    m1 = jnp.max(s, axis=-1, keepdims=True)
    i1 = jnp.min(jnp.where(s == m1, ei, _E), axis=-1, keepdims=True)
    s2 = jnp.where(ei == i1, -1.0, s)
    m2 = jnp.max(s2, axis=-1, keepdims=True)
    i2 = jnp.min(jnp.where(s2 == m2, ei, _E), axis=-1, keepdims=True)
    c1 = jnp.where(ei == i1, 1.0, 0.0)
    c2 = jnp.where(ei == i2, 1.0, 0.0)
    cc = c1 + c2

    ch_n = 256
    ti = lax.broadcasted_iota(jnp.int32, (ch_n, ch_n), 0)
    tj = lax.broadcasted_iota(jnp.int32, (ch_n, ch_n), 1)
    ltri = jnp.where(ti > tj, 1.0, 0.0)
    parts = []
    carry = jnp.zeros((1, _E), f32)
    for c in range(_T // ch_n):
        chk = lax.slice_in_dim(cc, c * ch_n, (c + 1) * ch_n, axis=0)
        parts.append(jnp.dot(ltri, chk, preferred_element_type=f32) + carry)
        carry = carry + jnp.sum(chk, axis=0, keepdims=True)
    excl = jnp.concatenate(parts, axis=0)
    hist = carry
    e8i = lax.broadcasted_iota(jnp.int32, (_E, _E), 0)
    e8j = lax.broadcasted_iota(jnp.int32, (_E, _E), 1)
    su8 = jnp.where(e8i < e8j, 1.0, 0.0)
    off = jnp.dot(hist, su8, preferred_element_type=f32, precision=lax.Precision.HIGHEST)

    offc = off + excl
    pos0f = jnp.sum(offc * c1, axis=-1, keepdims=True)
    pos1f = jnp.sum(offc * c2, axis=-1, keepdims=True)
    pos0 = pos0f.astype(jnp.int32)
    pos1 = pos1f.astype(jnp.int32)
    pos0_ref[...] = pos0
    pos1_ref[...] = pos1

    q_c = lax.broadcasted_iota(jnp.int32, (_NP, 1), 0)
    e_qc = q_c // _NT
    m_qc = q_c % _NT
    ohe_c = jnp.where(e_qc == lax.broadcasted_iota(jnp.int32, (_NP, _E), 1),
                      1.0, 0.0)
    lo_c = lax.dot_general(ohe_c, off, (((1,), (1,)), ((), ())),
                           preferred_element_type=f32, precision=lax.Precision.HIGHEST)
    hist_c = lax.dot_general(ohe_c, hist, (((1,), (1,)), ((), ())),
                             preferred_element_type=f32, precision=lax.Precision.HIGHEST)
    hi_c = lo_c + hist_c
    tlo_c = (m_qc * _TM).astype(f32)
    thi_c = tlo_c + _TM
    valid_c = jnp.where(
        (lo_c < thi_c) & (hi_c > tlo_c) & (hist_c > 0.5), 1.0, 0.0)
    slo_c = jnp.maximum(lo_c, tlo_c)
    shi_c = jnp.minimum(hi_c, thi_c)
    qi = lax.broadcasted_iota(jnp.int32, (_NP, _NP), 0)
    qj = lax.broadcasted_iota(jnp.int32, (_NP, _NP), 1)
    ltq = jnp.where(qi > qj, 1.0, 0.0)
    idq = jnp.where(qi == qj, 1.0, 0.0)
    rank_c = jnp.dot(ltq, valid_c, preferred_element_type=f32)
    rank_r = lax.dot_general(rank_c, idq, (((0,), (0,)), ((), ())),
                             preferred_element_type=f32)
    valid_r = lax.dot_general(valid_c, idq, (((0,), (0,)), ((), ())),
                              preferred_element_type=f32)
    ns = jnp.sum(valid_c)

    q0 = i1 * _NT + pos0 // _TM
    q1 = i2 * _NT + pos1 // _TM
    q_r = lax.broadcasted_iota(jnp.int32, (1, _NP), 1)
    oh0 = jnp.where(q0 == q_r, 1.0, 0.0)
    oh1 = jnp.where(q1 == q_r, 1.0, 0.0)
    rank0 = jnp.dot(oh0, rank_c, preferred_element_type=f32)
    rank1 = jnp.dot(oh1, rank_c, preferred_element_type=f32)
    spos0_ref[...] = rank0.astype(jnp.int32) * _TM + pos0 % _TM
    spos1_ref[...] = rank1.astype(jnp.int32) * _TM + pos1 % _TM

    g0_ref[...] = m1
    g1_ref[...] = m2

    g_col = lax.broadcasted_iota(jnp.int32, (128, 1), 0).astype(f32)
    g_cl = jnp.minimum(g_col, jnp.maximum(ns - 1.0, 0.0))
    sel = jnp.where((rank_r == g_cl) & (valid_r > 0.5), 1.0, 0.0)
    se_ref[...] = jnp.dot(sel, e_qc.astype(f32),
                          preferred_element_type=f32).astype(jnp.int32)
    sm_ref[...] = jnp.dot(sel, m_qc.astype(f32),
                          preferred_element_type=f32).astype(jnp.int32)
    slo_ref[...] = jnp.dot(sel, slo_c,
                           preferred_element_type=f32, precision=lax.Precision.HIGHEST).astype(jnp.int32)
    shi_ref[...] = jnp.dot(sel, shi_c,
                           preferred_element_type=f32, precision=lax.Precision.HIGHEST).astype(jnp.int32)


def _router(xf, wr):
    i32 = jnp.int32
    outs = pl.pallas_call(
        _router_body,
        in_specs=[pl.BlockSpec(memory_space=pltpu.VMEM),
                  pl.BlockSpec(memory_space=pltpu.VMEM)],
        out_shape=[
            jax.ShapeDtypeStruct((_T, 1), i32),
            jax.ShapeDtypeStruct((_T, 1), i32),
            jax.ShapeDtypeStruct((_T, 1), i32),
            jax.ShapeDtypeStruct((_T, 1), i32),
            jax.ShapeDtypeStruct((_T, 1), jnp.float32),
            jax.ShapeDtypeStruct((_T, 1), jnp.float32),
            jax.ShapeDtypeStruct((128, 1), i32),
            jax.ShapeDtypeStruct((128, 1), i32),
            jax.ShapeDtypeStruct((128, 1), i32),
            jax.ShapeDtypeStruct((128, 1), i32),
        ],
    )(xf, wr)
    return outs


def _gmm_body(se_ref, sm_ref, slo_ref, shi_ref,
              x_ref, w1_ref, w3_ref, w2_ref, o_ref):
    g = pl.program_id(0)
    lo = slo_ref[g]
    hi = shi_ref[g]
    m = sm_ref[g]
    r = m * _TM + lax.broadcasted_iota(jnp.int32, (_TM, 1), 0)
    valid = (r >= lo) & (r < hi)
    x = x_ref[...]
    a = jnp.dot(x, w1_ref[0], preferred_element_type=jnp.float32)
    b = jnp.dot(x, w3_ref[0], preferred_element_type=jnp.float32)
    h = (a * jax.nn.sigmoid(a)) * b
    h = jnp.where(valid, h, 0.0)
    o_ref[...] = jnp.dot(h, w2_ref[0], preferred_element_type=jnp.float32)


def _gmm(se, sm, slo, shi, xs, w1, w2, w3):
    grid_spec = pltpu.PrefetchScalarGridSpec(
        num_scalar_prefetch=4,
        grid=(_NS,),
        in_specs=[
            pl.BlockSpec((_TM, _DIM), lambda g, se, sm, slo, shi: (sm[g], 0)),
            pl.BlockSpec((1, _DIM, _HID),
                         lambda g, se, sm, slo, shi: (se[g], 0, 0)),
            pl.BlockSpec((1, _DIM, _HID),
                         lambda g, se, sm, slo, shi: (se[g], 0, 0)),
            pl.BlockSpec((1, _HID, _DIM),
                         lambda g, se, sm, slo, shi: (se[g], 0, 0)),
        ],
        out_specs=pl.BlockSpec((_TM, _DIM), lambda g, se, sm, slo, shi: (g, 0)),
    )
    return pl.pallas_call(
        _gmm_body,
        grid_spec=grid_spec,
        out_shape=jax.ShapeDtypeStruct((_NS * _TM, _DIM), jnp.float32),
        compiler_params=pltpu.CompilerParams(
            dimension_semantics=("arbitrary",),
        ),
    )(se, sm, slo, shi, xs, w1, w3, w2)


def _dispatch(xf, pos0, pos1):
    mesh = plsc.VectorSubcoreMesh(core_axis_name="c", subcore_axis_name="s")

    @functools.partial(
        pl.kernel,
        out_type=jax.ShapeDtypeStruct((_NA, _DIM), jnp.float32),
        mesh=mesh,
        scratch_types=[
            pltpu.VMEM((_TPW,), jnp.int32),
            pltpu.VMEM((_TPW,), jnp.int32),
            pltpu.VMEM((_TPW, _DIM), jnp.float32),
            pltpu.SemaphoreType.DMA,
            pltpu.SemaphoreType.DMA,
        ],
    )
    def body(x_hbm, p0_hbm, p1_hbm, xs_hbm, i0, i1, xv, sem0, sem1):
        wid = lax.axis_index("s") * 2 + lax.axis_index("c")
        t0 = wid * _TPW
        pltpu.sync_copy(p0_hbm.at[pl.ds(t0, _TPW)], i0)
        pltpu.sync_copy(p1_hbm.at[pl.ds(t0, _TPW)], i1)
        pltpu.sync_copy(x_hbm.at[pl.ds(t0, _TPW)], xv)
        c0 = pltpu.async_copy(xv, xs_hbm.at[i0], sem0)
        c1 = pltpu.async_copy(xv, xs_hbm.at[i1], sem1)
        c0.wait()
        c1.wait()

    return body(xf, pos0, pos1)


def _combine(ys, spos0, spos1, g0, g1):
    mesh = plsc.VectorSubcoreMesh(core_axis_name="c", subcore_axis_name="s")
    cw = 16
    nck = _TPW // cw

    @functools.partial(
        pl.kernel,
        out_type=jax.ShapeDtypeStruct((_T, _DIM), jnp.float32),
        mesh=mesh,
        scratch_types=[
            pltpu.VMEM((_TPW,), jnp.int32),
            pltpu.VMEM((_TPW,), jnp.int32),
            pltpu.VMEM((_TPW,), jnp.float32),
            pltpu.VMEM((_TPW,), jnp.float32),
            pltpu.VMEM((2, cw, _DIM), jnp.float32),
            pltpu.VMEM((2, cw, _DIM), jnp.float32),
            pltpu.VMEM((2, cw, _DIM), jnp.float32),
            pltpu.SemaphoreType.DMA((2,)),
            pltpu.SemaphoreType.DMA((2,)),
            pltpu.SemaphoreType.DMA((2,)),
        ],
    )
    def body(ys_hbm, sp0_hbm, sp1_hbm, g0_hbm, g1_hbm, o_hbm,
             i0, i1, gv0, gv1, b0, b1, oc, s0, s1, so):
        wid = lax.axis_index("s") * 2 + lax.axis_index("c")
        t0 = wid * _TPW
        pltpu.sync_copy(sp0_hbm.at[pl.ds(t0, _TPW)], i0)
        pltpu.sync_copy(sp1_hbm.at[pl.ds(t0, _TPW)], i1)
        pltpu.sync_copy(g0_hbm.at[pl.ds(t0, _TPW)], gv0)
        pltpu.sync_copy(g1_hbm.at[pl.ds(t0, _TPW)], gv1)

        def start_gather(c, sl):
            idx0 = i0[pl.ds(c * cw, cw)]
            idx1 = i1[pl.ds(c * cw, cw)]
            g0 = pltpu.make_async_copy(ys_hbm.at[idx0], b0.at[sl], s0.at[sl])
            g1 = pltpu.make_async_copy(ys_hbm.at[idx1], b1.at[sl], s1.at[sl])
            g0.start()
            g1.start()
            return g0, g1

        pend = start_gather(0, 0)
        wr = [None, None]
        for c in range(nck):
            sl = c % 2
            g0, g1 = pend
            if c + 1 < nck:
                nxt = start_gather(c + 1, (c + 1) % 2)
            g0.wait()
            g1.wait()
            if wr[sl] is not None:
                wr[sl].wait()
            wrow0 = gv0[pl.ds(c * cw, 16)]
            wrow1 = gv1[pl.ds(c * cw, 16)]
            for j in range(cw):
                w0 = wrow0[j]
                w1_ = wrow1[j]
                def add_body(k, _):
                    kb = k * 64
                    for u in range(4):
                        ds = pl.ds(kb + u * 16, 16)
                        oc[sl, j, ds] = w0 * b0[sl, j, ds] + w1_ * b1[sl, j, ds]
                    return 0
                lax.fori_loop(0, _DIM // 64, add_body, 0)
            w = pltpu.make_async_copy(
                oc.at[sl], o_hbm.at[pl.ds(t0 + c * cw, cw)], so.at[sl])
            w.start()
            wr[sl] = w
            if c + 1 < nck:
                pend = nxt
        for w in wr:
            if w is not None:
                w.wait()

    return body(ys, spos0, spos1, g0, g1)


def kernel(x, Wr, w1, w2, w3):
    bsz, seqlen, dim = x.shape
    xf = x.reshape(_T, _DIM)
    pos0, pos1, spos0, spos1, g0, g1, se, sm, slo, shi = _router(xf, Wr)
    pos0 = pos0.reshape(_T)
    pos1 = pos1.reshape(_T)
    spos0 = spos0.reshape(_T)
    spos1 = spos1.reshape(_T)
    g0 = g0.reshape(_T)
    g1 = g1.reshape(_T)
    se = se.reshape(128)
    sm = sm.reshape(128)
    slo = slo.reshape(128)
    shi = shi.reshape(128)
    xs = _dispatch(xf, pos0, pos1)
    ys = _gmm(se, sm, slo, shi, xs, w1, w2, w3)
    out = _combine(ys, spos0, spos1, g0, g1)
    return out.reshape(bsz, seqlen, dim)
